# baseline (speedup 1.0000x reference)
import dataclasses

import jax
import jax.numpy as jnp
from jax import lax
from jax.experimental import pallas as pl
from jax.experimental.pallas import tpu as pltpu
from jax.experimental.pallas import tpu_sc as plsc

LANES = 16
SPLITS = (0, 104)


def _compiler_params():
    cp = pltpu.CompilerParams(use_tc_tiling_on_sc=False)
    if "needs_layout_passes" in pltpu.CompilerParams.__dataclass_fields__:
        cp = dataclasses.replace(cp, needs_layout_passes=False)
    return cp


def _embed_kernel(b, l, embed):
    mesh = plsc.VectorSubcoreMesh(core_axis_name="core", subcore_axis_name="subcore")
    bounds = SPLITS + (l,)

    @jax.jit
    def run(idx, weights, aux):
        @pl.kernel(
            out_type=jax.ShapeDtypeStruct((b, l, embed), jnp.float32),
            mesh=mesh,
            compiler_params=_compiler_params(),
        )
        def kernel_body(w_hbm, i_hbm, aux_hbm, o_hbm):
            def body(i_vmem, aux_vmem, o_vmem):
                for s in range(len(SPLITS)):
                    lo, hi = bounds[s], bounds[s + 1]
                    pltpu.sync_copy(
                        w_hbm.at[i_vmem.at[0].at[pl.ds(lo, hi - lo)]],
                        o_vmem.at[0].at[pl.ds(lo, hi - lo)],
                    )

                def fixup(j, lane_lo):
                    v = i_vmem[0, pl.ds(j, LANES)]
                    m = v == jnp.int32(1)
                    if lane_lo:
                        m &= lax.broadcasted_iota(jnp.int32, (LANES,), 0) >= lane_lo

                    @pl.when(jnp.any(m))
                    def _():
                        rows = j + lax.broadcasted_iota(jnp.int32, (LANES,), 0)
                        zeros = jnp.zeros((LANES,), jnp.int32)
                        for cb in range(embed // LANES):
                            av = aux_vmem[1, pl.ds(cb * LANES, LANES)]
                            for t in range(LANES):
                                c = cb * LANES + t
                                plsc.addupdate_scatter(
                                    o_vmem,
                                    [zeros, rows, jnp.full((LANES,), c, jnp.int32)],
                                    jnp.full((LANES,), av[t], jnp.float32),
                                    mask=m,
                                )

                @pl.loop(0, (l // LANES) * LANES, step=LANES)
                def _(j):
                    fixup(j, 0)

                if l % LANES:
                    fixup(l - LANES, LANES - l % LANES)

            pltpu.emit_pipeline(
                body,
                grid=(b,),
                in_specs=[
                    pl.BlockSpec((1, l), index_map=lambda i: (i, 0)),
                    pl.BlockSpec((2, embed), index_map=lambda i: (0, 0)),
                ],
                out_specs=[
                    pl.BlockSpec((1, l, embed), index_map=lambda i: (i, 0, 0)),
                ],
                core_axis_name=("core", "subcore"),
                dimension_semantics=(pltpu.PARALLEL,),
            )(i_hbm, aux_hbm, o_hbm)

        return kernel_body(weights, idx, aux)

    return run


def kernel(input_, weights, aux):
    b, l = input_.shape
    idx = jnp.asarray(input_, jnp.int32)
    return _embed_kernel(b, l, weights.shape[1])(idx, weights, aux)


# speedup vs baseline: 1.1142x; 1.1142x over previous
import dataclasses
import functools

import jax
import jax.numpy as jnp
from jax import lax
from jax.experimental import pallas as pl
from jax.experimental.pallas import tpu as pltpu
from jax.experimental.pallas import tpu_sc as plsc

LANES = 16
SPLITS = (0, 104)


def _compiler_params():
    cp = pltpu.CompilerParams(use_tc_tiling_on_sc=False)
    if "needs_layout_passes" in pltpu.CompilerParams.__dataclass_fields__:
        cp = dataclasses.replace(cp, needs_layout_passes=False)
    return cp


BB = 4


def _embed_kernel(b, l, embed):
    mesh = plsc.VectorSubcoreMesh(core_axis_name="core", subcore_axis_name="subcore")
    bounds = SPLITS + (l,)

    @jax.jit
    def run(idx, weights, aux):
        @pl.kernel(
            out_type=jax.ShapeDtypeStruct((b, l, embed), jnp.float32),
            mesh=mesh,
            scratch_types=[pltpu.SemaphoreType.DMA],
            compiler_params=_compiler_params(),
        )
        def kernel_body(w_hbm, i_hbm, aux_hbm, o_hbm, sem):
            def body(i_vmem, aux_vmem, o_vmem):
                copies = []
                for r in range(BB):
                    for s in range(len(SPLITS)):
                        lo, hi = bounds[s], bounds[s + 1]
                        copies.append(pltpu.async_copy(
                            w_hbm.at[i_vmem.at[r].at[pl.ds(lo, hi - lo)]],
                            o_vmem.at[r].at[pl.ds(lo, hi - lo)],
                            sem,
                        ))
                for c in copies:
                    c.wait()

                def fixup(r, j, lane_lo):
                    v = i_vmem[r, pl.ds(j, LANES)]
                    m = v == jnp.int32(1)
                    if lane_lo:
                        m &= lax.broadcasted_iota(jnp.int32, (LANES,), 0) >= lane_lo

                    @pl.when(jnp.any(m))
                    def _():
                        rows = j + lax.broadcasted_iota(jnp.int32, (LANES,), 0)
                        rfull = jnp.full((LANES,), r, jnp.int32)
                        for cb in range(embed // LANES):
                            av = aux_vmem[1, pl.ds(cb * LANES, LANES)]
                            for t in range(LANES):
                                c = cb * LANES + t
                                plsc.addupdate_scatter(
                                    o_vmem,
                                    [rfull, rows, jnp.full((LANES,), c, jnp.int32)],
                                    jnp.full((LANES,), av[t], jnp.float32),
                                    mask=m,
                                )

                for r in range(BB):
                    @pl.loop(0, (l // LANES) * LANES, step=LANES)
                    def _(j, r=r):
                        fixup(r, j, 0)

                    if l % LANES:
                        fixup(r, l - LANES, LANES - l % LANES)

            pltpu.emit_pipeline(
                body,
                grid=(b // BB,),
                in_specs=[
                    pl.BlockSpec((BB, l), index_map=lambda i: (i, 0)),
                    pl.BlockSpec((2, embed), index_map=lambda i: (0, 0)),
                ],
                out_specs=[
                    pl.BlockSpec((BB, l, embed), index_map=lambda i: (i, 0, 0)),
                ],
                core_axis_name=("core", "subcore"),
                dimension_semantics=(pltpu.PARALLEL,),
            )(i_hbm, aux_hbm, o_hbm)

        return kernel_body(weights, idx, aux)

    return run


def kernel(input_, weights, aux):
    b, l = input_.shape
    idx = jnp.asarray(input_, jnp.int32)
    return _embed_kernel(b, l, weights.shape[1])(idx, weights, aux)
